# Initial kernel scaffold; baseline (speedup 1.0000x reference)
import jax
import jax.numpy as jnp
from jax.experimental import pallas as pl


def kernel(query, knowledge_embed, knowledge_full):
    raise NotImplementedError("write your pallas kernel here")


# speedup vs baseline: 5.3503x; 5.3503x over previous
import functools

import jax
import jax.numpy as jnp
from jax.experimental import pallas as pl
from jax.experimental.pallas import tpu as pltpu
from jax.experimental.pallas import tpu_sc as plsc

TOPK = 16
EPS = 1e-8
NEG_INF = float("-inf")
INT_MAX = 0x7FFFFFFF
CHUNK = 128


def _stage_a_kernel(nb, kblk, ktotal, qn_ref, kn_ref, sims_ref, cid_ref, cmax_ref):
    j = pl.program_id(0)
    q = qn_ref.shape[0]
    nchunk_blk = kblk // CHUNK

    s = jax.lax.dot_general(
        qn_ref[...], kn_ref[...],
        (((1,), (1,)), ((), ())),
        preferred_element_type=jnp.float32,
    )
    sims_ref[...] = s

    def chunk_maxes(sblk):
        cms = [
            jnp.max(sblk[:, g * CHUNK : (g + 1) * CHUNK], axis=1, keepdims=True)
            for g in range(nchunk_blk)
        ]
        return jnp.transpose(jnp.concatenate(cms, axis=1))

    cmax_ref[pl.ds(j * nchunk_blk, nchunk_blk), :] = chunk_maxes(s)

    @pl.when(j == nb - 1)
    def _select_chunks():
        col = jax.lax.broadcasted_iota(jnp.int32, (q, kblk), 1)
        sm = jnp.where(col + j * kblk < ktotal, s, NEG_INF)
        cmax_ref[pl.ds(j * nchunk_blk, nchunk_blk), :] = chunk_maxes(sm)
        nchunk = nb * nchunk_blk
        c = cmax_ref[...]
        row = jax.lax.broadcasted_iota(jnp.int32, (nchunk, q), 0)
        ids = []
        for _ in range(TOPK):
            m = jnp.max(c, axis=0, keepdims=True)
            cand = jnp.where(c == m, row, nchunk)
            a = jnp.min(cand, axis=0, keepdims=True)
            ids.append(a)
            c = jnp.where(row == a, NEG_INF, c)
        cid_ref[...] = jnp.concatenate(ids, axis=0)


def _stage_a(qn, kn, kblk=2048):
    q, d = qn.shape
    ktotal = kn.shape[0]
    nb = pl.cdiv(ktotal, kblk)
    nchunk = nb * (kblk // CHUNK)
    return pl.pallas_call(
        functools.partial(_stage_a_kernel, nb, kblk, ktotal),
        grid=(nb,),
        in_specs=[
            pl.BlockSpec((q, d), lambda j: (0, 0)),
            pl.BlockSpec((kblk, d), lambda j: (j, 0)),
        ],
        out_specs=[
            pl.BlockSpec((q, kblk), lambda j: (0, j)),
            pl.BlockSpec((TOPK, q), lambda j: (0, 0)),
        ],
        out_shape=[
            jax.ShapeDtypeStruct((q, nb * kblk), jnp.float32),
            jax.ShapeDtypeStruct((TOPK, q), jnp.int32),
        ],
        scratch_shapes=[pltpu.VMEM((nchunk, q), jnp.float32)],
    )(qn, kn)


def _stage_c_kernel(ktotal, s_ref, cid_ref, oi_ref):
    q = s_ref.shape[0]
    s = s_ref[...]
    cid = cid_ref[...]
    loc = jax.lax.broadcasted_iota(jnp.int32, (q, CHUNK), 1)
    gcol = jnp.concatenate(
        [cid[:, r : r + 1] * CHUNK + loc for r in range(TOPK)], axis=1
    )
    s = jnp.where(gcol < ktotal, s, NEG_INF)
    outs = []
    for _ in range(TOPK):
        m = jnp.max(s, axis=1, keepdims=True)
        cand = jnp.where(s == m, gcol, INT_MAX)
        a = jnp.min(cand, axis=1, keepdims=True)
        outs.append(a)
        s = jnp.where(gcol == a, NEG_INF, s)
    oi_ref[...] = jnp.concatenate(outs, axis=1)


def _stage_c(cands, cid, ktotal):
    q = cands.shape[0]
    return pl.pallas_call(
        functools.partial(_stage_c_kernel, ktotal),
        out_shape=jax.ShapeDtypeStruct((q, TOPK), jnp.int32),
    )(cands, cid)


def _sc_gather1(src, flat_idx):
    n = flat_idx.shape[0]
    window = 128
    idx2 = flat_idx.reshape(1, n)
    width = src.shape[1]

    @pl.kernel(
        out_type=jax.ShapeDtypeStruct((n, width), src.dtype),
        mesh=plsc.VectorSubcoreMesh(core_axis_name="core", subcore_axis_name="subcore"),
    )
    def kern(src_hbm, i_hbm, o_hbm):
        def body(i_vmem, o_vmem):
            pltpu.sync_copy(src_hbm.at[i_vmem.at[0]], o_vmem)

        pltpu.emit_pipeline(
            body,
            grid=(n // window,),
            in_specs=[pl.BlockSpec((1, window), index_map=lambda i: (0, i))],
            out_specs=[pl.BlockSpec((window, width), index_map=lambda i: (i, 0))],
            core_axis_name=("core", "subcore"),
            dimension_semantics=(pltpu.PARALLEL,),
        )(i_hbm, o_hbm)

    return kern(src, idx2)


def _sc_gather2(full, embed, flat_idx):
    n = flat_idx.shape[0]
    window = 128
    idx2 = flat_idx.reshape(1, n)
    lfull = full.shape[1]
    lembed = embed.shape[1]

    @pl.kernel(
        out_type=[
            jax.ShapeDtypeStruct((n, lfull), full.dtype),
            jax.ShapeDtypeStruct((n, lembed), embed.dtype),
        ],
        mesh=plsc.VectorSubcoreMesh(core_axis_name="core", subcore_axis_name="subcore"),
    )
    def kern(full_hbm, embed_hbm, i_hbm, of_hbm, oe_hbm):
        def body(i_vmem, of_vmem, oe_vmem):
            pltpu.sync_copy(full_hbm.at[i_vmem.at[0]], of_vmem)
            pltpu.sync_copy(embed_hbm.at[i_vmem.at[0]], oe_vmem)

        pltpu.emit_pipeline(
            body,
            grid=(n // window,),
            in_specs=[pl.BlockSpec((1, window), index_map=lambda i: (0, i))],
            out_specs=[
                pl.BlockSpec((window, lfull), index_map=lambda i: (i, 0)),
                pl.BlockSpec((window, lembed), index_map=lambda i: (i, 0)),
            ],
            core_axis_name=("core", "subcore"),
            dimension_semantics=(pltpu.PARALLEL,),
        )(i_hbm, of_hbm, oe_hbm)

    return kern(full, embed, idx2)


def kernel(query, knowledge_embed, knowledge_full):
    qn = query / jnp.maximum(jnp.linalg.norm(query, axis=-1, keepdims=True), EPS)
    kn = knowledge_embed / jnp.maximum(
        jnp.linalg.norm(knowledge_embed, axis=-1, keepdims=True), EPS
    )
    q = query.shape[0]
    ktotal = knowledge_embed.shape[0]

    sims, cid_t = _stage_a(qn, kn)
    kpad = sims.shape[1]
    nchunk = kpad // CHUNK
    cid = cid_t.T

    rowids = (cid + jnp.arange(q, dtype=jnp.int32)[:, None] * nchunk).reshape(-1)
    cands = _sc_gather1(sims.reshape(q * nchunk, CHUNK), rowids)
    cands = cands.reshape(q, TOPK * CHUNK)

    indices = _stage_c(cands, cid, ktotal)
    flat = indices.reshape(q * TOPK)

    lfull = knowledge_full.shape[1]
    pad = (-lfull) % 128
    kf = jnp.pad(knowledge_full, ((0, 0), (0, pad))) if pad else knowledge_full
    topk_knowledge, topk_embed = _sc_gather2(kf, knowledge_embed, flat)
    return (
        topk_knowledge.reshape(q, TOPK, lfull + pad)[:, :, :lfull],
        topk_embed.reshape(q, TOPK, knowledge_embed.shape[1]),
    )
